# baseline (speedup 1.0000x reference)
import functools

import jax
import jax.numpy as jnp
from jax import lax
from jax.experimental import pallas as pl
from jax.experimental.pallas import tpu as pltpu
from jax.experimental.pallas import tpu_sc as plsc

B = 4096
S = 200
E = 32
L = 16

NC = 2
NS = 16
NW = NC * NS

SEQ_PER_WORKER = B // NW
GCHUNK = 50
ROWS_PER_SEQ = S // GCHUNK
K = 4
ROWS_PER_CHUNK = K * ROWS_PER_SEQ
IDX_PER_CHUNK = K * S
CHUNKS = SEQ_PER_WORKER // K
IDX_ROWS_PER_WORKER = SEQ_PER_WORKER * ROWS_PER_SEQ


def _fire_gathers(tok_hbm, idx_all, rows_v, gsem, g):
    for j in range(ROWS_PER_CHUNK):
        pltpu.make_async_copy(
            tok_hbm.at[idx_all.at[g * ROWS_PER_CHUNK + j]],
            rows_v.at[pl.ds(j * GCHUNK, GCHUNK)],
            gsem,
        ).start()


def _drain(hbm_dummy, vmem_ref, sem):
    pltpu.make_async_copy(hbm_dummy, vmem_ref, sem).wait()


def _add_positions(rows_v, pos_v):
    def add_body(s, c2):
        p0 = pos_v[s, pl.ds(0, L)]
        p1 = pos_v[s, pl.ds(L, L)]
        for k in range(K):
            r = k * S + s
            rows_v[r, pl.ds(0, L)] = rows_v[r, pl.ds(0, L)] + p0
            rows_v[r, pl.ds(L, L)] = rows_v[r, pl.ds(L, L)] + p1
        return c2

    lax.fori_loop(0, S, add_body, 0, unroll=2)


def _sc_body(seq_hbm, tok_hbm, pos_hbm, out_hbm,
             idx_all, rows0, rows1, pos_v, gsem0, gsem1, osem0, osem1):
    wid = lax.axis_index("s") * NC + lax.axis_index("c")
    rows = (rows0, rows1)
    gsems = (gsem0, gsem1)
    osems = (osem0, osem1)
    out_worker_base = wid * (SEQ_PER_WORKER * S)

    pltpu.sync_copy(pos_hbm, pos_v)
    pltpu.sync_copy(
        seq_hbm.at[pl.ds(wid * IDX_ROWS_PER_WORKER, IDX_ROWS_PER_WORKER)],
        idx_all)

    def fire_bf(g, buf):
        for j in range(ROWS_PER_CHUNK):
            pltpu.make_async_copy(
                tok_hbm.at[idx_all.at[g * ROWS_PER_CHUNK + j]],
                rows[buf].at[pl.ds(j * GCHUNK, GCHUNK)],
                gsems[buf],
            ).start()

    fire_bf(0, 0)

    def outer(gg, carry):
        for b in (0, 1):
            g = gg * 2 + b
            nb = 1 - b
            _drain(tok_hbm.at[pl.ds(0, IDX_PER_CHUNK)], rows[b], gsems[b])

            @pl.when(g + 1 < CHUNKS)
            def _():
                fire_bf(g + 1, nb)
        return carry

    lax.fori_loop(0, CHUNKS // 2, outer, 0)


@jax.jit
def _sc_embed(seq2, token_table, pos_table):
    mesh = plsc.VectorSubcoreMesh(
        core_axis_name="c", subcore_axis_name="s", num_cores=NC, num_subcores=NS
    )
    return pl.kernel(
        _sc_body,
        out_type=jax.ShapeDtypeStruct((B * S, E), jnp.float32),
        mesh=mesh,
        compiler_params=pltpu.CompilerParams(use_tc_tiling_on_sc=False),
        scratch_types=[
            pltpu.VMEM((IDX_ROWS_PER_WORKER, GCHUNK), jnp.int32),
            pltpu.VMEM((IDX_PER_CHUNK, E), jnp.bfloat16),
            pltpu.VMEM((IDX_PER_CHUNK, E), jnp.bfloat16),
            pltpu.VMEM((S, E), jnp.float32),
            pltpu.SemaphoreType.DMA,
            pltpu.SemaphoreType.DMA,
            pltpu.SemaphoreType.DMA,
            pltpu.SemaphoreType.DMA,
        ],
    )(seq2, token_table, pos_table)


def kernel(seq, token_table, pos_table):
    seq2 = seq.reshape(B * S // GCHUNK, GCHUNK).astype(jnp.int32)
    out = _sc_embed(seq2, token_table.astype(jnp.bfloat16), pos_table)
    return out.reshape(B, S, E)


# speedup vs baseline: 1.0714x; 1.0714x over previous
import functools

import jax
import jax.numpy as jnp
from jax import lax
from jax.experimental import pallas as pl
from jax.experimental.pallas import tpu as pltpu
from jax.experimental.pallas import tpu_sc as plsc

B = 4096
S = 200
E = 32
L = 16

NC = 2
NS = 16
NW = NC * NS

SEQ_PER_WORKER = B // NW
SPLITS = ((0, 104), (104, 96))
K = 4
CHUNKS = SEQ_PER_WORKER // K
CHUNK_BYTES = K * S * E * 4


def _fire_gathers(tok_hbm, idx_all, rows_v, gsem, g):
    for k in range(K):
        for off, ln in SPLITS:
            pltpu.make_async_copy(
                tok_hbm.at[idx_all.at[g * K + k, pl.ds(off, ln)]],
                rows_v.at[k, pl.ds(off, ln)],
                gsem,
            ).start()


def _drain(hbm_dummy, vmem_ref, sem):
    pltpu.make_async_copy(hbm_dummy, vmem_ref, sem).wait()


def _add_positions(rows_v, pos_v):
    def add_body(s, c2):
        p0 = pos_v[s, pl.ds(0, L)]
        p1 = pos_v[s, pl.ds(L, L)]
        for k in range(K):
            rows_v[k, s, pl.ds(0, L)] = rows_v[k, s, pl.ds(0, L)] + p0
            rows_v[k, s, pl.ds(L, L)] = rows_v[k, s, pl.ds(L, L)] + p1
        return c2

    lax.fori_loop(0, S, add_body, 0, unroll=2)


def _sc_body(seq_hbm, tok_hbm, pos_hbm, out_hbm,
             idx_all, rows0, rows1, pos_v, gsem0, gsem1, osem0, osem1):
    wid = lax.axis_index("s") * NC + lax.axis_index("c")
    rows = (rows0, rows1)
    gsems = (gsem0, gsem1)
    osems = (osem0, osem1)
    seq_base = wid * SEQ_PER_WORKER

    pltpu.sync_copy(pos_hbm, pos_v)
    pltpu.sync_copy(seq_hbm.at[pl.ds(seq_base, SEQ_PER_WORKER)], idx_all)

    _fire_gathers(tok_hbm, idx_all, rows[0], gsems[0], 0)

    def outer(gg, carry):
        for b in (0, 1):
            g = gg * 2 + b
            nb = 1 - b
            _drain(out_hbm.at[pl.ds(0, K)], rows[b], gsems[b])

            @pl.when(g >= 1)
            def _():
                _drain(out_hbm.at[pl.ds(0, K)], rows[nb], osems[nb])

            @pl.when(g + 1 < CHUNKS)
            def _():
                _fire_gathers(tok_hbm, idx_all, rows[nb], gsems[nb], g + 1)

            _add_positions(rows[b], pos_v)

            pltpu.make_async_copy(
                rows[b],
                out_hbm.at[pl.ds(seq_base + g * K, K)],
                osems[b],
            ).start()
        return carry

    lax.fori_loop(0, CHUNKS // 2, outer, 0)

    _drain(out_hbm.at[pl.ds(0, K)], rows[(CHUNKS - 1) % 2],
           osems[(CHUNKS - 1) % 2])


@jax.jit
def _sc_embed(seq, token_table, pos_table):
    mesh = plsc.VectorSubcoreMesh(
        core_axis_name="c", subcore_axis_name="s", num_cores=NC, num_subcores=NS
    )
    return pl.kernel(
        _sc_body,
        out_type=jax.ShapeDtypeStruct((B, S, E), jnp.float32),
        mesh=mesh,
        compiler_params=pltpu.CompilerParams(use_tc_tiling_on_sc=False),
        scratch_types=[
            pltpu.VMEM((SEQ_PER_WORKER, S), jnp.int32),
            pltpu.VMEM((K, S, E), jnp.float32),
            pltpu.VMEM((K, S, E), jnp.float32),
            pltpu.VMEM((S, E), jnp.float32),
            pltpu.SemaphoreType.DMA,
            pltpu.SemaphoreType.DMA,
            pltpu.SemaphoreType.DMA,
            pltpu.SemaphoreType.DMA,
        ],
    )(seq, token_table, pos_table)


def kernel(seq, token_table, pos_table):
    return _sc_embed(seq, token_table, pos_table)
